# baseline (speedup 1.0000x reference)
import functools

import jax
import jax.numpy as jnp
from jax import lax
from jax.experimental import pallas as pl
from jax.experimental.pallas import tpu as pltpu
from jax.experimental.pallas import tpu_sc as plsc

_NC, _NS = 2, 16
_NW = _NC * _NS
_CHUNK = 128


def _sc_gather(idx2d, table):
    num_chunks, chunk = idx2d.shape
    d = table.shape[1]
    cpw = num_chunks // _NW
    rpg = 4
    mesh = plsc.VectorSubcoreMesh(core_axis_name="c", subcore_axis_name="s")

    @functools.partial(
        pl.kernel,
        mesh=mesh,
        out_type=jax.ShapeDtypeStruct((num_chunks, chunk, d), jnp.float32),
        scratch_types=[
            pltpu.VMEM((cpw, chunk), jnp.int32),
            pltpu.VMEM((rpg, chunk, d), jnp.float32),
            pltpu.SemaphoreType.DMA,
        ],
    )
    def gather_kernel(idx_hbm, table_hbm, out_hbm, idx_v, rows_v, sem):
        wid = lax.axis_index("s") * _NC + lax.axis_index("c")
        base = wid * cpw
        pltpu.sync_copy(idx_hbm.at[pl.ds(base, cpw)], idx_v)
        for r in range(cpw // rpg):
            copies = [
                pltpu.async_copy(
                    table_hbm.at[idx_v.at[r * rpg + k]], rows_v.at[k], sem
                )
                for k in range(rpg)
            ]
            for c in copies:
                c.wait()
            pltpu.sync_copy(rows_v, out_hbm.at[pl.ds(base + r * rpg, rpg)])

    return gather_kernel(idx2d, table)


def _tc_head(x, pos_tiled, W, b2, targets2, block_rows):
    bt, d = x.shape
    v = W.shape[1]
    steps = bt // block_rows
    inv_n = 1.0 / bt

    def body(x_ref, pos_ref, w_ref, b_ref, t_ref, logits_ref, loss_ref):
        i = pl.program_id(0)
        xp = x_ref[...] + pos_ref[...]
        logits = (
            jnp.dot(xp, w_ref[...], preferred_element_type=jnp.float32) + b_ref[...]
        )
        logits_ref[...] = logits
        rowmax = jnp.max(logits, axis=1, keepdims=True)
        sumexp = jnp.sum(jnp.exp(logits - rowmax), axis=1, keepdims=True)
        lse = rowmax + jnp.log(sumexp)
        colid = lax.broadcasted_iota(jnp.int32, (block_rows, v), 1)
        tmask = colid == t_ref[...]
        tlogit = jnp.sum(
            jnp.where(tmask, logits, 0.0), axis=1, keepdims=True
        )
        partial = jnp.sum(lse - tlogit, axis=0, keepdims=True) * inv_n

        @pl.when(i == 0)
        def _():
            loss_ref[...] = jnp.zeros_like(loss_ref)

        loss_ref[...] += partial

    return pl.pallas_call(
        body,
        grid=(steps,),
        in_specs=[
            pl.BlockSpec((block_rows, d), lambda i: (i, 0)),
            pl.BlockSpec((block_rows, d), lambda i: (0, 0)),
            pl.BlockSpec((d, v), lambda i: (0, 0)),
            pl.BlockSpec((1, v), lambda i: (0, 0)),
            pl.BlockSpec((block_rows, 1), lambda i: (i, 0)),
        ],
        out_specs=[
            pl.BlockSpec((block_rows, v), lambda i: (i, 0)),
            pl.BlockSpec((1, 1), lambda i: (0, 0)),
        ],
        out_shape=[
            jax.ShapeDtypeStruct((bt, v), jnp.float32),
            jax.ShapeDtypeStruct((1, 1), jnp.float32),
        ],
    )(x, pos_tiled, W, b2, targets2)


def kernel(idx, targets, tok_table, pos_table, W, b):
    B, T = idx.shape
    bt = B * T
    d = tok_table.shape[1]
    v = W.shape[1]
    dp = 128
    block_rows = 1024

    idx2d = idx.reshape(bt // _CHUNK, _CHUNK).astype(jnp.int32)
    table_p = jnp.pad(tok_table, ((0, 0), (0, dp - d)))
    x = _sc_gather(idx2d, table_p).reshape(bt, dp)

    pos_tiled = jnp.pad(
        jnp.tile(pos_table, (block_rows // T, 1)), ((0, 0), (0, dp - d))
    )
    w_p = jnp.pad(W, ((0, dp - d), (0, 0)))
    b2 = b.reshape(1, v)
    targets2 = targets.reshape(bt, 1).astype(jnp.int32)

    logits_flat, loss_acc = _tc_head(x, pos_tiled, w_p, b2, targets2, block_rows)
    return logits_flat.reshape(B, T, v), loss_acc[0, 0]


# speedup vs baseline: 1.8632x; 1.8632x over previous
import functools

import jax
import jax.numpy as jnp
from jax import lax
from jax.experimental import pallas as pl
from jax.experimental.pallas import tpu as pltpu
from jax.experimental.pallas import tpu_sc as plsc

_NC, _NS = 2, 16
_NW = _NC * _NS
_CHUNK = 128


def _sc_gather(idx2d, table):
    num_chunks, chunk = idx2d.shape
    d = table.shape[1]
    cpw = num_chunks // _NW
    rpg = 4
    mesh = plsc.VectorSubcoreMesh(core_axis_name="c", subcore_axis_name="s")

    @functools.partial(
        pl.kernel,
        mesh=mesh,
        out_type=jax.ShapeDtypeStruct((num_chunks, chunk, d), jnp.float32),
        scratch_types=[
            pltpu.VMEM((cpw, chunk), jnp.int32),
            pltpu.VMEM((rpg, chunk, d), jnp.float32),
            pltpu.SemaphoreType.DMA,
        ],
    )
    def gather_kernel(idx_hbm, table_hbm, out_hbm, idx_v, rows_v, sem):
        wid = lax.axis_index("s") * _NC + lax.axis_index("c")
        base = wid * cpw
        pltpu.sync_copy(idx_hbm.at[pl.ds(base, cpw)], idx_v)
        for r in range(cpw // rpg):
            copies = [
                pltpu.async_copy(
                    table_hbm.at[idx_v.at[r * rpg + k]], rows_v.at[k], sem
                )
                for k in range(rpg)
            ]
            for c in copies:
                c.wait()
            pltpu.sync_copy(rows_v, out_hbm.at[pl.ds(base + r * rpg, rpg)])

    return gather_kernel(idx2d, table)


def _tc_head(xt3, posp, wt, b2, t_t, block_b):
    t_dim, b_dim, dp = xt3.shape
    v = wt.shape[0]
    steps = b_dim // block_b
    inv_n = 1.0 / (t_dim * b_dim)

    def body(x_ref, pos_ref, w_ref, b_ref, t_ref, out_ref, loss_ref):
        i = pl.program_id(0)
        partials = []
        for t in range(t_dim):
            xp = x_ref[t] + pos_ref[t, :][None, :]
            lg = (
                lax.dot_general(
                    w_ref[...],
                    xp,
                    (((1,), (1,)), ((), ())),
                    preferred_element_type=jnp.float32,
                )
                + b_ref[...]
            )
            out_ref[t] = lg
            mx = jnp.max(lg, axis=0, keepdims=True)
            se = jnp.sum(jnp.exp(lg - mx), axis=0, keepdims=True)
            lse = mx + jnp.log(se)
            rowid = lax.broadcasted_iota(jnp.int32, (v, block_b), 0)
            tmask = rowid == t_ref[t, :][None, :]
            tl = jnp.sum(jnp.where(tmask, lg, 0.0), axis=0, keepdims=True)
            partials.append(jnp.sum(lse - tl, axis=1, keepdims=True))
        partial = sum(partials) * inv_n

        @pl.when(i == 0)
        def _():
            loss_ref[...] = jnp.zeros_like(loss_ref)

        loss_ref[...] += partial

    return pl.pallas_call(
        body,
        grid=(steps,),
        in_specs=[
            pl.BlockSpec((t_dim, block_b, dp), lambda i: (0, i, 0)),
            pl.BlockSpec((t_dim, dp), lambda i: (0, 0)),
            pl.BlockSpec((v, dp), lambda i: (0, 0)),
            pl.BlockSpec((v, 1), lambda i: (0, 0)),
            pl.BlockSpec((t_dim, block_b), lambda i: (0, i)),
        ],
        out_specs=[
            pl.BlockSpec((t_dim, v, block_b), lambda i: (0, 0, i)),
            pl.BlockSpec((1, 1), lambda i: (0, 0)),
        ],
        out_shape=[
            jax.ShapeDtypeStruct((t_dim, v, b_dim), jnp.float32),
            jax.ShapeDtypeStruct((1, 1), jnp.float32),
        ],
    )(xt3, posp, wt, b2, t_t)


def kernel(idx, targets, tok_table, pos_table, W, b):
    B, T = idx.shape
    bt = B * T
    d = tok_table.shape[1]
    v = W.shape[1]
    dp = 128
    block_b = 128

    idx2d = idx.T.reshape(bt // _CHUNK, _CHUNK).astype(jnp.int32)
    table_p = jnp.pad(tok_table, ((0, 0), (0, dp - d)))
    xt3 = _sc_gather(idx2d, table_p).reshape(T, B, dp)

    posp = jnp.pad(pos_table, ((0, 0), (0, dp - d)))
    wt = jnp.pad(W.T, ((0, 0), (0, dp - d)))
    b2 = b.reshape(v, 1)
    t_t = targets.T.astype(jnp.int32)

    out3, loss_acc = _tc_head(xt3, posp, wt, b2, t_t, block_b)
    return jnp.transpose(out3, (2, 0, 1)), loss_acc[0, 0]


# speedup vs baseline: 2.0958x; 1.1248x over previous
import functools

import jax
import jax.numpy as jnp
from jax import lax
from jax.experimental import pallas as pl
from jax.experimental.pallas import tpu as pltpu
from jax.experimental.pallas import tpu_sc as plsc

_NC, _NS = 2, 16
_NW = _NC * _NS
_CHUNK = 128


def _sc_gather(idx2d, table):
    num_chunks, chunk = idx2d.shape
    d = table.shape[1]
    cpw = num_chunks // _NW
    rpg = 4
    mesh = plsc.VectorSubcoreMesh(core_axis_name="c", subcore_axis_name="s")

    @functools.partial(
        pl.kernel,
        mesh=mesh,
        out_type=jax.ShapeDtypeStruct((num_chunks, chunk, d), jnp.float32),
        scratch_types=[
            pltpu.VMEM((cpw, chunk), jnp.int32),
            pltpu.VMEM((rpg, chunk, d), jnp.float32),
            pltpu.SemaphoreType.DMA,
        ],
    )
    def gather_kernel(idx_hbm, table_hbm, out_hbm, idx_v, rows_v, sem):
        wid = lax.axis_index("s") * _NC + lax.axis_index("c")
        base = wid * cpw
        pltpu.sync_copy(idx_hbm.at[pl.ds(base, cpw)], idx_v)
        for r in range(cpw // rpg):
            copies = [
                pltpu.async_copy(
                    table_hbm.at[idx_v.at[r * rpg + k]], rows_v.at[k], sem
                )
                for k in range(rpg)
            ]
            for c in copies:
                c.wait()
            pltpu.sync_copy(rows_v, out_hbm.at[pl.ds(base + r * rpg, rpg)])

    return gather_kernel(idx2d, table)


def _tc_head(xt3, posp, wt, b2, t_t, block_b):
    t_dim, b_dim, dp = xt3.shape
    v = wt.shape[0]
    steps = b_dim // block_b
    inv_n = 1.0 / (t_dim * b_dim)

    def body(x_ref, pos_ref, w_ref, b_ref, t_ref, out_ref, loss_ref):
        i = pl.program_id(0)
        partials = []
        for t in range(t_dim):
            xp = x_ref[t] + pos_ref[t, :][None, :]
            lg = (
                lax.dot_general(
                    w_ref[...],
                    xp,
                    (((1,), (1,)), ((), ())),
                    preferred_element_type=jnp.float32,
                )
                + b_ref[...]
            )
            out_ref[t] = lg
            mx = jnp.max(lg, axis=0, keepdims=True)
            se = jnp.sum(jnp.exp(lg - mx), axis=0, keepdims=True)
            lse = mx + jnp.log(se)
            rowid = lax.broadcasted_iota(jnp.int32, (v, block_b), 0)
            tmask = rowid == t_ref[t, :][None, :]
            tl = jnp.sum(jnp.where(tmask, lg, 0.0), axis=0, keepdims=True)
            partials.append(jnp.sum(lse - tl, axis=1, keepdims=True))
        partial = sum(partials) * inv_n

        @pl.when(i == 0)
        def _():
            loss_ref[...] = jnp.zeros_like(loss_ref)

        loss_ref[...] += partial

    return pl.pallas_call(
        body,
        grid=(steps,),
        in_specs=[
            pl.BlockSpec((t_dim, block_b, dp), lambda i: (0, i, 0)),
            pl.BlockSpec((t_dim, dp), lambda i: (0, 0)),
            pl.BlockSpec((v, dp), lambda i: (0, 0)),
            pl.BlockSpec((v, 1), lambda i: (0, 0)),
            pl.BlockSpec((t_dim, block_b), lambda i: (0, i)),
        ],
        out_specs=[
            pl.BlockSpec((t_dim, v, block_b), lambda i: (0, 0, i)),
            pl.BlockSpec((1, 1), lambda i: (0, 0)),
        ],
        out_shape=[
            jax.ShapeDtypeStruct((t_dim, v, b_dim), jnp.float32),
            jax.ShapeDtypeStruct((1, 1), jnp.float32),
        ],
    )(xt3, posp, wt, b2, t_t)


def kernel(idx, targets, tok_table, pos_table, W, b):
    B, T = idx.shape
    bt = B * T
    d = tok_table.shape[1]
    v = W.shape[1]
    dp = 128
    block_b = 256

    idx2d = idx.T.reshape(bt // _CHUNK, _CHUNK).astype(jnp.int32)
    table_p = jnp.pad(tok_table, ((0, 0), (0, dp - d)))
    xt3 = _sc_gather(idx2d, table_p).reshape(T, B, dp)

    posp = jnp.pad(pos_table, ((0, 0), (0, dp - d)))
    wt = jnp.pad(W.T, ((0, 0), (0, dp - d)))
    b2 = b.reshape(v, 1)
    t_t = targets.T.astype(jnp.int32)

    out3, loss_acc = _tc_head(xt3, posp, wt, b2, t_t, block_b)
    return jnp.transpose(out3, (2, 0, 1)), loss_acc[0, 0]
